# baseline (speedup 1.0000x reference)
import jax
import jax.numpy as jnp
from jax import lax
from jax.experimental import pallas as pl
from jax.experimental.pallas import tpu as pltpu
from jax.experimental.pallas import tpu_sc as plsc

ROWS = 1000000
DIM = 64
N_CHUNKS = 4
BATCH = 16384
PRIME = (1 << 31) - 1
VROWS = ROWS // 2
VDIM = 2 * DIM

NC = 2
NS = 16
LANES = 16
NW = NC * NS
B_PER_W = BATCH // NW
PIECE = 32
N_PIECE = B_PER_W // PIECE
SCALE = (N_CHUNKS * DIM) ** 0.5 / N_CHUNKS

W_COLS = 256
N_WIN = ROWS // W_COLS
W_PER_T = (N_WIN + NW - 1) // NW

_U = jnp.uint32


def _modp(r):
    return jnp.minimum(r, r - _U(PRIME))


def _shift_modp(v, k):
    lo = (v << _U(k)) & _U(PRIME)
    hi = v >> _U(31 - k)
    return _modp(lo + hi)


def _poly_hash_vec(xv, a2, a1, a0, bv):
    t2 = _shift_modp(_modp(a2 * xv), 22)
    t1 = _shift_modp(_modp(a1 * xv), 11)
    t0 = _modp(a0 * xv)
    s = _modp(t2 + t1)
    s = _modp(s + t0)
    return _modp(s + bv)


def _fmt_body(tT_hbm, fmt_hbm, in_v, out_v, tin_v, tout_v,
              sem_i0, sem_i1, sem_o0, sem_o1):
    wid = lax.axis_index("s") * NC + lax.axis_index("c")
    lane = lax.iota(jnp.int32, LANES)
    rows_jb = [jnp.int32(16 * q) + lane for q in range(4)]
    sem_i = [sem_i0, sem_i1]
    sem_o = [sem_o0, sem_o1]

    def col_of(j):
        c = (wid + jnp.int32(32) * j) * jnp.int32(W_COLS)
        return pl.multiple_of(c, W_COLS)

    def row_of(j):
        return pl.multiple_of(col_of(j) >> jnp.int32(1), W_COLS // 2)

    def fire_in(j, bb):
        return pltpu.async_copy(
            tT_hbm.at[:, pl.ds(col_of(j), W_COLS)],
            in_v.at[bb, :, pl.ds(0, W_COLS)], sem_i[bb])

    def transpose_window(bb, nrows):
        bbv = jnp.full((LANES,), bb, jnp.int32)

        def tr(i, _):
            c0 = jnp.full((LANES,), 2 * i, jnp.int32)
            c1 = c0 + jnp.int32(1)
            for jb in range(8):
                col = c1 if jb >= 4 else c0
                v = plsc.load_gather(in_v, [bbv, rows_jb[jb % 4], col])
                out_v[bb, i, pl.ds(16 * jb, 16)] = v
            return _

        lax.fori_loop(jnp.int32(0), jnp.int32(nrows), tr, jnp.int32(0))

    def wait_in(bb):
        pltpu.make_async_copy(
            tT_hbm.at[:, pl.ds(0, W_COLS)],
            in_v.at[bb, :, pl.ds(0, W_COLS)], sem_i[bb]).wait()

    def wait_out(bb):
        pltpu.make_async_copy(
            out_v.at[bb], fmt_hbm.at[pl.ds(0, W_COLS // 2)],
            sem_o[bb]).wait()

    NJ = W_PER_T - 1
    fire_in(jnp.int32(0), 0)
    fire_in(jnp.int32(1), 1)

    def pair(g, carry):
        for bb in range(2):
            j = jnp.int32(2) * g + jnp.int32(bb)
            wait_in(bb)

            @pl.when(g > 0)
            def _wo(bb=bb):
                wait_out(bb)

            transpose_window(bb, W_COLS // 2)
            pltpu.async_copy(
                out_v.at[bb], fmt_hbm.at[pl.ds(row_of(j), W_COLS // 2)],
                sem_o[bb])

            @pl.when(j < jnp.int32(NJ - 2))
            def _fi(j=j, bb=bb):
                fire_in(j + jnp.int32(2), bb)
        return carry

    lax.fori_loop(jnp.int32(0), jnp.int32(NJ // 2), pair, jnp.int32(0))
    wait_out(0)

    @pl.when(wid < 2)
    def _do_last():
        j = jnp.int32(W_PER_T - 1)
        cp = fire_in(j, 0)
        cp.wait()
        transpose_window(0, W_COLS // 2)
        pltpu.async_copy(
            out_v.at[0], fmt_hbm.at[pl.ds(row_of(j), W_COLS // 2)],
            sem_o[0]).wait()

    wait_out(1)

    @pl.when(wid == NW - 1)
    def _do_tail():
        pltpu.sync_copy(tT_hbm.at[:, pl.ds(ROWS - DIM, DIM)], tin_v)

        def tr(i, _):
            c0 = jnp.full((LANES,), 2 * i, jnp.int32)
            c1 = c0 + jnp.int32(1)
            for jb in range(8):
                col = c1 if jb >= 4 else c0
                v = plsc.load_gather(tin_v, [rows_jb[jb % 4], col])
                tout_v[i, pl.ds(16 * jb, 16)] = v
            return _

        lax.fori_loop(jnp.int32(0), jnp.int32(DIM // 2), tr, jnp.int32(0))
        pltpu.sync_copy(tout_v, fmt_hbm.at[pl.ds(VROWS - DIM // 2, DIM // 2)])


@jax.jit
def _format(tT):
    mesh = plsc.VectorSubcoreMesh(core_axis_name="c", subcore_axis_name="s")
    f = pl.kernel(
        _fmt_body,
        out_type=jax.ShapeDtypeStruct((VROWS, VDIM), jnp.float32),
        mesh=mesh,
        scratch_types=[
            pltpu.VMEM((2, DIM, W_COLS + 1), jnp.float32),
            pltpu.VMEM((2, W_COLS // 2, VDIM), jnp.float32),
            pltpu.VMEM((DIM, DIM), jnp.float32),
            pltpu.VMEM((DIM // 2, VDIM), jnp.float32),
            pltpu.SemaphoreType.DMA,
            pltpu.SemaphoreType.DMA,
            pltpu.SemaphoreType.DMA,
            pltpu.SemaphoreType.DMA,
        ],
        compiler_params=pltpu.CompilerParams(needs_layout_passes=False),
    )
    return f(tT)


def _sc_body(x_hbm, view_hbm, params_hbm, out_hbm,
             x_v, params_v, h0r_v, h0c_v, h1r_v, h1c_v, w_v, rows_v,
             wrows_v, out_v, sem_r0, sem_r1, sem_o0, sem_o1):
    wid = lax.axis_index("s") * NC + lax.axis_index("c")
    base = wid * B_PER_W

    pltpu.sync_copy(x_hbm.at[pl.ds(base, B_PER_W)], x_v)
    pltpu.sync_copy(params_hbm, params_v)

    coef = [[[params_v[f, c, j] for j in range(4)]
             for c in range(N_CHUNKS)] for f in range(2)]

    def hash_piece(p):
        def body(j, _):
            xv = x_v[pl.ds(p * PIECE + j * LANES, LANES)].astype(_U)
            for c in range(N_CHUNKS):
                c0 = coef[0][c]
                h0 = _poly_hash_vec(xv, c0[0].astype(_U), c0[1].astype(_U),
                                    c0[2].astype(_U), c0[3].astype(_U))
                h0 = h0 % _U(ROWS)
                c1 = coef[1][c]
                h1 = _poly_hash_vec(xv, c1[0].astype(_U), c1[1].astype(_U),
                                    c1[2].astype(_U), c1[3].astype(_U))
                h1 = h1 % _U(ROWS * DIM)
                sl = pl.ds(j * LANES, LANES)
                h0r_v[c, p, sl] = (h0 >> _U(1)).astype(jnp.int32)
                h0c_v[c, p, sl] = ((h0 & _U(1)) << _U(6)).astype(jnp.int32)
                h1r_v[c, p, sl] = (h1 >> _U(7)).astype(jnp.int32)
                h1c_v[c, p, sl] = (h1 & _U(VDIM - 1)).astype(jnp.int32)
            return _

        lax.fori_loop(jnp.int32(0), jnp.int32(PIECE // LANES), body,
                      jnp.int32(0))

    for p in range(N_PIECE):
        hash_piece(p)

    sem_r = [sem_r0, sem_r1]
    sem_o = [sem_o0, sem_o1]

    def fire(p):
        buf = p % 2
        cps = []
        for c in range(N_CHUNKS):
            cps.append(pltpu.async_copy(view_hbm.at[h0r_v.at[c, p]],
                                        rows_v.at[buf, c], sem_r[buf]))
            cps.append(pltpu.async_copy(view_hbm.at[h1r_v.at[c, p]],
                                        wrows_v.at[buf, c], sem_r[buf]))
        return cps

    cps = fire(0)
    out_cps = [None, None]
    lane = lax.iota(jnp.int32, LANES)
    for p in range(N_PIECE):
        buf = p % 2
        for cp in cps:
            cp.wait()
        if p + 1 < N_PIECE:
            cps = fire(p + 1)
        if out_cps[buf] is not None:
            out_cps[buf].wait()
            out_cps[buf] = None

        bufv = jnp.full((LANES,), buf, jnp.int32)
        for c in range(N_CHUNKS):
            cv = jnp.full((LANES,), c, jnp.int32)
            for g in range(PIECE // LANES):
                cols = h1c_v[c, p, pl.ds(g * LANES, LANES)]
                wv = plsc.load_gather(
                    wrows_v, [bufv, cv, lane + jnp.int32(g * LANES), cols])
                w_v[c, pl.ds(p * PIECE + g * LANES, LANES)] = wv * SCALE

        pv = jnp.full((LANES,), p, jnp.int32)

        def body(b, _):
            bidx = jnp.full((LANES,), p * PIECE + b, jnp.int32)
            bv_ = jnp.full((LANES,), b, jnp.int32)
            cvs = [jnp.full((LANES,), c, jnp.int32) for c in range(N_CHUNKS)]
            wv = [plsc.load_gather(w_v, [cvs[c], bidx])
                  for c in range(N_CHUNKS)]
            cb = [plsc.load_gather(h0c_v, [cvs[c], pv, bv_]) + lane
                  for c in range(N_CHUNKS)]
            for d in range(DIM // LANES):
                sl = pl.ds(d * LANES, LANES)
                acc = None
                for c in range(N_CHUNKS):
                    rv = plsc.load_gather(
                        rows_v,
                        [bufv, cvs[c], bv_, cb[c] + jnp.int32(d * LANES)])
                    acc = wv[c] * rv if acc is None else acc + wv[c] * rv
                out_v[buf, b, sl] = acc
            return _

        lax.fori_loop(jnp.int32(0), jnp.int32(PIECE), body, jnp.int32(0))
        out_cps[buf] = pltpu.async_copy(
            out_v.at[buf], out_hbm.at[pl.ds(base + p * PIECE, PIECE)],
            sem_o[buf])

    for cp in out_cps:
        if cp is not None:
            cp.wait()


@jax.jit
def _wh_embed(x_i32, view, params):
    mesh = plsc.VectorSubcoreMesh(core_axis_name="c", subcore_axis_name="s")
    f = pl.kernel(
        _sc_body,
        out_type=jax.ShapeDtypeStruct((BATCH, DIM), jnp.float32),
        mesh=mesh,
        scratch_types=[
            pltpu.VMEM((B_PER_W,), jnp.int32),
            pltpu.VMEM((2, N_CHUNKS, 4, LANES), jnp.int32),
            pltpu.VMEM((N_CHUNKS, N_PIECE, PIECE), jnp.int32),
            pltpu.VMEM((N_CHUNKS, N_PIECE, PIECE), jnp.int32),
            pltpu.VMEM((N_CHUNKS, N_PIECE, PIECE), jnp.int32),
            pltpu.VMEM((N_CHUNKS, N_PIECE, PIECE), jnp.int32),
            pltpu.VMEM((N_CHUNKS, B_PER_W), jnp.float32),
            pltpu.VMEM((2, N_CHUNKS, PIECE, VDIM), jnp.float32),
            pltpu.VMEM((2, N_CHUNKS, PIECE, VDIM), jnp.float32),
            pltpu.VMEM((2, PIECE, DIM), jnp.float32),
            pltpu.SemaphoreType.DMA,
            pltpu.SemaphoreType.DMA,
            pltpu.SemaphoreType.DMA,
            pltpu.SemaphoreType.DMA,
        ],
        compiler_params=pltpu.CompilerParams(needs_layout_passes=False),
    )
    return f(x_i32, view, params)


def kernel(x, table, a0, b0, a1, b1):
    x_i32 = x.astype(jnp.int32)
    p = []
    for a, b in ((a0, b0), (a1, b1)):
        a = a.astype(jnp.int64)
        p.append(jnp.stack([a >> 22, (a >> 11) & 2047, a & 2047,
                            b.astype(jnp.int64)], axis=-1))
    params = jnp.stack(p).astype(jnp.int32)
    params = jnp.broadcast_to(params[..., None],
                              (2, N_CHUNKS, 4, LANES))
    with jax.enable_x64(False):
        view = _format(jnp.swapaxes(table, 0, 1))
        return _wh_embed(x_i32, view, params)


# speedup vs baseline: 1.0021x; 1.0021x over previous
import jax
import jax.numpy as jnp
from jax import lax
from jax.experimental import pallas as pl
from jax.experimental.pallas import tpu as pltpu
from jax.experimental.pallas import tpu_sc as plsc

ROWS = 1000000
DIM = 64
N_CHUNKS = 4
BATCH = 16384
PRIME = (1 << 31) - 1
VROWS = ROWS // 2
VDIM = 2 * DIM

NC = 2
NS = 16
LANES = 16
NW = NC * NS
B_PER_W = BATCH // NW
PIECE = 32
N_PIECE = B_PER_W // PIECE
SCALE = (N_CHUNKS * DIM) ** 0.5 / N_CHUNKS

W_COLS = 256
N_WIN = ROWS // W_COLS
W_PER_T = (N_WIN + NW - 1) // NW

_U = jnp.uint32


def _modp(r):
    return jnp.minimum(r, r - _U(PRIME))


def _shift_modp(v, k):
    lo = (v << _U(k)) & _U(PRIME)
    hi = v >> _U(31 - k)
    return _modp(lo + hi)


def _poly_hash_vec(xv, a2, a1, a0, bv):
    t2 = _shift_modp(_modp(a2 * xv), 22)
    t1 = _shift_modp(_modp(a1 * xv), 11)
    t0 = _modp(a0 * xv)
    s = _modp(t2 + t1)
    s = _modp(s + t0)
    return _modp(s + bv)


def _fmt_body(tT_hbm, fmt_hbm, in0_v, in1_v, out_v, tin_v, tout_v,
              sem_i0, sem_i1, sem_o0, sem_o1):
    wid = lax.axis_index("s") * NC + lax.axis_index("c")
    lane = lax.iota(jnp.int32, LANES)
    rows_jb = [jnp.int32(16 * q) + lane for q in range(4)]
    sem_i = [sem_i0, sem_i1]
    sem_o = [sem_o0, sem_o1]

    def col_of(j):
        c = (wid + jnp.int32(32) * j) * jnp.int32(W_COLS)
        return pl.multiple_of(c, W_COLS)

    def row_of(j):
        return pl.multiple_of(col_of(j) >> jnp.int32(1), W_COLS // 2)

    in_bufs = [in0_v, in1_v]

    def fire_in(j, bb):
        return pltpu.async_copy(
            tT_hbm.at[:, pl.ds(col_of(j), W_COLS)],
            in_bufs[bb].at[:, pl.ds(0, W_COLS)], sem_i[bb])

    def transpose_window(bb, nrows):
        src = in_bufs[bb]

        def tr(i, _):
            c0 = jnp.full((LANES,), 2 * i, jnp.int32)
            c1 = c0 + jnp.int32(1)
            for jb in range(8):
                col = c1 if jb >= 4 else c0
                v = plsc.load_gather(src, [rows_jb[jb % 4], col])
                out_v[bb, i, pl.ds(16 * jb, 16)] = v
            return _

        lax.fori_loop(jnp.int32(0), jnp.int32(nrows), tr, jnp.int32(0))

    def wait_in(bb):
        pltpu.make_async_copy(
            tT_hbm.at[:, pl.ds(0, W_COLS)],
            in_bufs[bb].at[:, pl.ds(0, W_COLS)], sem_i[bb]).wait()

    def wait_out(bb):
        pltpu.make_async_copy(
            out_v.at[bb], fmt_hbm.at[pl.ds(0, W_COLS // 2)],
            sem_o[bb]).wait()

    NJ = W_PER_T - 1
    fire_in(jnp.int32(0), 0)
    fire_in(jnp.int32(1), 1)

    def pair(g, carry):
        for bb in range(2):
            j = jnp.int32(2) * g + jnp.int32(bb)
            wait_in(bb)

            @pl.when(g > 0)
            def _wo(bb=bb):
                wait_out(bb)

            transpose_window(bb, W_COLS // 2)
            pltpu.async_copy(
                out_v.at[bb], fmt_hbm.at[pl.ds(row_of(j), W_COLS // 2)],
                sem_o[bb])

            @pl.when(j < jnp.int32(NJ - 2))
            def _fi(j=j, bb=bb):
                fire_in(j + jnp.int32(2), bb)
        return carry

    lax.fori_loop(jnp.int32(0), jnp.int32(NJ // 2), pair, jnp.int32(0))
    wait_out(0)

    @pl.when(wid < 2)
    def _do_last():
        j = jnp.int32(W_PER_T - 1)
        cp = fire_in(j, 0)
        cp.wait()
        transpose_window(0, W_COLS // 2)
        pltpu.async_copy(
            out_v.at[0], fmt_hbm.at[pl.ds(row_of(j), W_COLS // 2)],
            sem_o[0]).wait()

    wait_out(1)

    @pl.when(wid == NW - 1)
    def _do_tail():
        pltpu.sync_copy(tT_hbm.at[:, pl.ds(ROWS - DIM, DIM)], tin_v)

        def tr(i, _):
            c0 = jnp.full((LANES,), 2 * i, jnp.int32)
            c1 = c0 + jnp.int32(1)
            for jb in range(8):
                col = c1 if jb >= 4 else c0
                v = plsc.load_gather(tin_v, [rows_jb[jb % 4], col])
                tout_v[i, pl.ds(16 * jb, 16)] = v
            return _

        lax.fori_loop(jnp.int32(0), jnp.int32(DIM // 2), tr, jnp.int32(0))
        pltpu.sync_copy(tout_v, fmt_hbm.at[pl.ds(VROWS - DIM // 2, DIM // 2)])


@jax.jit
def _format(tT):
    mesh = plsc.VectorSubcoreMesh(core_axis_name="c", subcore_axis_name="s")
    f = pl.kernel(
        _fmt_body,
        out_type=jax.ShapeDtypeStruct((VROWS, VDIM), jnp.float32),
        mesh=mesh,
        scratch_types=[
            pltpu.VMEM((DIM, W_COLS + 1), jnp.float32),
            pltpu.VMEM((DIM, W_COLS + 1), jnp.float32),
            pltpu.VMEM((2, W_COLS // 2, VDIM), jnp.float32),
            pltpu.VMEM((DIM, DIM), jnp.float32),
            pltpu.VMEM((DIM // 2, VDIM), jnp.float32),
            pltpu.SemaphoreType.DMA,
            pltpu.SemaphoreType.DMA,
            pltpu.SemaphoreType.DMA,
            pltpu.SemaphoreType.DMA,
        ],
        compiler_params=pltpu.CompilerParams(needs_layout_passes=False,
                                             disable_bounds_checks=True),
    )
    return f(tT)


def _sc_body(x_hbm, view_hbm, params_hbm, out_hbm,
             x_v, params_v, h0r_v, h0c_v, h1r_v, h1c_v, w_v, rows_v,
             wrows_v, out_v, sem_r0, sem_r1, sem_o0, sem_o1):
    wid = lax.axis_index("s") * NC + lax.axis_index("c")
    base = wid * B_PER_W

    pltpu.sync_copy(x_hbm.at[pl.ds(base, B_PER_W)], x_v)
    pltpu.sync_copy(params_hbm, params_v)

    coef = [[[params_v[f, c, j] for j in range(4)]
             for c in range(N_CHUNKS)] for f in range(2)]

    def hash_piece(p):
        def body(j, _):
            xv = x_v[pl.ds(p * PIECE + j * LANES, LANES)].astype(_U)
            for c in range(N_CHUNKS):
                c0 = coef[0][c]
                h0 = _poly_hash_vec(xv, c0[0].astype(_U), c0[1].astype(_U),
                                    c0[2].astype(_U), c0[3].astype(_U))
                h0 = h0 % _U(ROWS)
                c1 = coef[1][c]
                h1 = _poly_hash_vec(xv, c1[0].astype(_U), c1[1].astype(_U),
                                    c1[2].astype(_U), c1[3].astype(_U))
                h1 = h1 % _U(ROWS * DIM)
                sl = pl.ds(j * LANES, LANES)
                h0r_v[c, p, sl] = (h0 >> _U(1)).astype(jnp.int32)
                h0c_v[c, p, sl] = ((h0 & _U(1)) << _U(6)).astype(jnp.int32)
                h1r_v[c, p, sl] = (h1 >> _U(7)).astype(jnp.int32)
                h1c_v[c, p, sl] = (h1 & _U(VDIM - 1)).astype(jnp.int32)
            return _

        lax.fori_loop(jnp.int32(0), jnp.int32(PIECE // LANES), body,
                      jnp.int32(0))

    for p in range(N_PIECE):
        hash_piece(p)

    sem_r = [sem_r0, sem_r1]
    sem_o = [sem_o0, sem_o1]

    def fire(p):
        buf = p % 2
        cps = []
        for c in range(N_CHUNKS):
            cps.append(pltpu.async_copy(view_hbm.at[h0r_v.at[c, p]],
                                        rows_v.at[buf, c], sem_r[buf]))
            cps.append(pltpu.async_copy(view_hbm.at[h1r_v.at[c, p]],
                                        wrows_v.at[buf, c], sem_r[buf]))
        return cps

    cps = fire(0)
    out_cps = [None, None]
    lane = lax.iota(jnp.int32, LANES)
    for p in range(N_PIECE):
        buf = p % 2
        for cp in cps:
            cp.wait()
        if p + 1 < N_PIECE:
            cps = fire(p + 1)
        if out_cps[buf] is not None:
            out_cps[buf].wait()
            out_cps[buf] = None

        bufv = jnp.full((LANES,), buf, jnp.int32)
        for c in range(N_CHUNKS):
            cv = jnp.full((LANES,), c, jnp.int32)
            for g in range(PIECE // LANES):
                cols = h1c_v[c, p, pl.ds(g * LANES, LANES)]
                wv = plsc.load_gather(
                    wrows_v, [bufv, cv, lane + jnp.int32(g * LANES), cols])
                w_v[c, pl.ds(p * PIECE + g * LANES, LANES)] = wv * SCALE

        pv = jnp.full((LANES,), p, jnp.int32)

        def body(b, _):
            bidx = jnp.full((LANES,), p * PIECE + b, jnp.int32)
            bv_ = jnp.full((LANES,), b, jnp.int32)
            cvs = [jnp.full((LANES,), c, jnp.int32) for c in range(N_CHUNKS)]
            wv = [plsc.load_gather(w_v, [cvs[c], bidx])
                  for c in range(N_CHUNKS)]
            cb = [plsc.load_gather(h0c_v, [cvs[c], pv, bv_]) + lane
                  for c in range(N_CHUNKS)]
            for d in range(DIM // LANES):
                sl = pl.ds(d * LANES, LANES)
                acc = None
                for c in range(N_CHUNKS):
                    rv = plsc.load_gather(
                        rows_v,
                        [bufv, cvs[c], bv_, cb[c] + jnp.int32(d * LANES)])
                    acc = wv[c] * rv if acc is None else acc + wv[c] * rv
                out_v[buf, b, sl] = acc
            return _

        lax.fori_loop(jnp.int32(0), jnp.int32(PIECE), body, jnp.int32(0))
        out_cps[buf] = pltpu.async_copy(
            out_v.at[buf], out_hbm.at[pl.ds(base + p * PIECE, PIECE)],
            sem_o[buf])

    for cp in out_cps:
        if cp is not None:
            cp.wait()


@jax.jit
def _wh_embed(x_i32, view, params):
    mesh = plsc.VectorSubcoreMesh(core_axis_name="c", subcore_axis_name="s")
    f = pl.kernel(
        _sc_body,
        out_type=jax.ShapeDtypeStruct((BATCH, DIM), jnp.float32),
        mesh=mesh,
        scratch_types=[
            pltpu.VMEM((B_PER_W,), jnp.int32),
            pltpu.VMEM((2, N_CHUNKS, 4, LANES), jnp.int32),
            pltpu.VMEM((N_CHUNKS, N_PIECE, PIECE), jnp.int32),
            pltpu.VMEM((N_CHUNKS, N_PIECE, PIECE), jnp.int32),
            pltpu.VMEM((N_CHUNKS, N_PIECE, PIECE), jnp.int32),
            pltpu.VMEM((N_CHUNKS, N_PIECE, PIECE), jnp.int32),
            pltpu.VMEM((N_CHUNKS, B_PER_W), jnp.float32),
            pltpu.VMEM((2, N_CHUNKS, PIECE, VDIM), jnp.float32),
            pltpu.VMEM((2, N_CHUNKS, PIECE, VDIM), jnp.float32),
            pltpu.VMEM((2, PIECE, DIM), jnp.float32),
            pltpu.SemaphoreType.DMA,
            pltpu.SemaphoreType.DMA,
            pltpu.SemaphoreType.DMA,
            pltpu.SemaphoreType.DMA,
        ],
        compiler_params=pltpu.CompilerParams(needs_layout_passes=False),
    )
    return f(x_i32, view, params)


def kernel(x, table, a0, b0, a1, b1):
    x_i32 = x.astype(jnp.int32)
    p = []
    for a, b in ((a0, b0), (a1, b1)):
        a = a.astype(jnp.int64)
        p.append(jnp.stack([a >> 22, (a >> 11) & 2047, a & 2047,
                            b.astype(jnp.int64)], axis=-1))
    params = jnp.stack(p).astype(jnp.int32)
    params = jnp.broadcast_to(params[..., None],
                              (2, N_CHUNKS, 4, LANES))
    with jax.enable_x64(False):
        view = _format(jnp.swapaxes(table, 0, 1))
        return _wh_embed(x_i32, view, params)


# speedup vs baseline: 1.3375x; 1.3346x over previous
import jax
import jax.numpy as jnp
from jax import lax
from jax.experimental import pallas as pl
from jax.experimental.pallas import tpu as pltpu
from jax.experimental.pallas import tpu_sc as plsc

ROWS = 1000000
DIM = 64
N_CHUNKS = 4
BATCH = 16384
PRIME = (1 << 31) - 1
VROWS = ROWS // 2
VDIM = 2 * DIM

NC = 2
NS = 16
LANES = 16
NW = NC * NS
B_PER_W = BATCH // NW
PIECE = 32
N_PIECE = B_PER_W // PIECE
SCALE = (N_CHUNKS * DIM) ** 0.5 / N_CHUNKS

W_COLS = 256
N_WIN = ROWS // W_COLS
W_PER_T = (N_WIN + NW - 1) // NW

_U = jnp.uint32


def _modp(r):
    return jnp.minimum(r, r - _U(PRIME))


def _shift_modp(v, k):
    lo = (v << _U(k)) & _U(PRIME)
    hi = v >> _U(31 - k)
    return _modp(lo + hi)


def _poly_hash_vec(xv, a2, a1, a0, bv):
    t2 = _shift_modp(_modp(a2 * xv), 22)
    t1 = _shift_modp(_modp(a1 * xv), 11)
    t0 = _modp(a0 * xv)
    s = _modp(t2 + t1)
    s = _modp(s + t0)
    return _modp(s + bv)


def _fmt_body(tT_hbm, fmt_hbm, in0_v, in1_v, out_v, tin_v, tout_v,
              sem_i0, sem_i1, sem_o0, sem_o1):
    wid = lax.axis_index("s") * NC + lax.axis_index("c")
    lane = lax.iota(jnp.int32, LANES)
    rows_jb = [jnp.int32(16 * q) + lane for q in range(4)]
    sem_i = [sem_i0, sem_i1]
    sem_o = [sem_o0, sem_o1]

    def col_of(j):
        c = (wid + jnp.int32(32) * j) * jnp.int32(W_COLS)
        return pl.multiple_of(c, W_COLS)

    def row_of(j):
        return pl.multiple_of(col_of(j) >> jnp.int32(1), W_COLS // 2)

    in_bufs = [in0_v, in1_v]

    def fire_in(j, bb):
        return pltpu.async_copy(
            tT_hbm.at[:, pl.ds(col_of(j), W_COLS)],
            in_bufs[bb].at[:, pl.ds(0, W_COLS)], sem_i[bb])

    def transpose_window(bb, nrows):
        src = in_bufs[bb]
        UNROLL = 4

        def tr(iq, _):
            i0 = iq * jnp.int32(UNROLL)
            vs = []
            for u in range(UNROLL):
                c0 = jnp.full((LANES,), 2 * (i0 + jnp.int32(u)), jnp.int32)
                c1 = c0 + jnp.int32(1)
                for jb in range(8):
                    col = c1 if jb >= 4 else c0
                    vs.append(plsc.load_gather(src, [rows_jb[jb % 4], col]))
            for u in range(UNROLL):
                for jb in range(8):
                    out_v[bb, i0 + jnp.int32(u), pl.ds(16 * jb, 16)] = (
                        vs[u * 8 + jb])
            return _

        lax.fori_loop(jnp.int32(0), jnp.int32(nrows // UNROLL), tr,
                      jnp.int32(0))

    def wait_in(bb):
        pltpu.make_async_copy(
            tT_hbm.at[:, pl.ds(0, W_COLS)],
            in_bufs[bb].at[:, pl.ds(0, W_COLS)], sem_i[bb]).wait()

    def wait_out(bb):
        pltpu.make_async_copy(
            out_v.at[bb], fmt_hbm.at[pl.ds(0, W_COLS // 2)],
            sem_o[bb]).wait()

    NJ = W_PER_T - 1
    fire_in(jnp.int32(0), 0)
    fire_in(jnp.int32(1), 1)

    def pair(g, carry):
        for bb in range(2):
            j = jnp.int32(2) * g + jnp.int32(bb)
            wait_in(bb)

            @pl.when(g > 0)
            def _wo(bb=bb):
                wait_out(bb)

            transpose_window(bb, W_COLS // 2)
            pltpu.async_copy(
                out_v.at[bb], fmt_hbm.at[pl.ds(row_of(j), W_COLS // 2)],
                sem_o[bb])

            @pl.when(j < jnp.int32(NJ - 2))
            def _fi(j=j, bb=bb):
                fire_in(j + jnp.int32(2), bb)
        return carry

    lax.fori_loop(jnp.int32(0), jnp.int32(NJ // 2), pair, jnp.int32(0))
    wait_out(0)

    @pl.when(wid < 2)
    def _do_last():
        j = jnp.int32(W_PER_T - 1)
        cp = fire_in(j, 0)
        cp.wait()
        transpose_window(0, W_COLS // 2)
        pltpu.async_copy(
            out_v.at[0], fmt_hbm.at[pl.ds(row_of(j), W_COLS // 2)],
            sem_o[0]).wait()

    wait_out(1)

    @pl.when(wid == NW - 1)
    def _do_tail():
        pltpu.sync_copy(tT_hbm.at[:, pl.ds(ROWS - DIM, DIM)], tin_v)

        def tr(i, _):
            c0 = jnp.full((LANES,), 2 * i, jnp.int32)
            c1 = c0 + jnp.int32(1)
            for jb in range(8):
                col = c1 if jb >= 4 else c0
                v = plsc.load_gather(tin_v, [rows_jb[jb % 4], col])
                tout_v[i, pl.ds(16 * jb, 16)] = v
            return _

        lax.fori_loop(jnp.int32(0), jnp.int32(DIM // 2), tr, jnp.int32(0))
        pltpu.sync_copy(tout_v, fmt_hbm.at[pl.ds(VROWS - DIM // 2, DIM // 2)])


@jax.jit
def _format(tT):
    mesh = plsc.VectorSubcoreMesh(core_axis_name="c", subcore_axis_name="s")
    f = pl.kernel(
        _fmt_body,
        out_type=jax.ShapeDtypeStruct((VROWS, VDIM), jnp.float32),
        mesh=mesh,
        scratch_types=[
            pltpu.VMEM((DIM, W_COLS + 1), jnp.float32),
            pltpu.VMEM((DIM, W_COLS + 1), jnp.float32),
            pltpu.VMEM((2, W_COLS // 2, VDIM), jnp.float32),
            pltpu.VMEM((DIM, DIM), jnp.float32),
            pltpu.VMEM((DIM // 2, VDIM), jnp.float32),
            pltpu.SemaphoreType.DMA,
            pltpu.SemaphoreType.DMA,
            pltpu.SemaphoreType.DMA,
            pltpu.SemaphoreType.DMA,
        ],
        compiler_params=pltpu.CompilerParams(needs_layout_passes=False,
                                             disable_bounds_checks=True),
    )
    return f(tT)


def _sc_body(x_hbm, view_hbm, params_hbm, out_hbm,
             x_v, params_v, h0r_v, h0c_v, h1r_v, h1c_v, w_v, rows_v,
             wrows_v, out_v, sem_r0, sem_r1, sem_o0, sem_o1):
    wid = lax.axis_index("s") * NC + lax.axis_index("c")
    base = wid * B_PER_W

    pltpu.sync_copy(x_hbm.at[pl.ds(base, B_PER_W)], x_v)
    pltpu.sync_copy(params_hbm, params_v)

    coef = [[[params_v[f, c, j] for j in range(4)]
             for c in range(N_CHUNKS)] for f in range(2)]

    def hash_piece(p):
        def body(j, _):
            xv = x_v[pl.ds(p * PIECE + j * LANES, LANES)].astype(_U)
            for c in range(N_CHUNKS):
                c0 = coef[0][c]
                h0 = _poly_hash_vec(xv, c0[0].astype(_U), c0[1].astype(_U),
                                    c0[2].astype(_U), c0[3].astype(_U))
                h0 = h0 % _U(ROWS)
                c1 = coef[1][c]
                h1 = _poly_hash_vec(xv, c1[0].astype(_U), c1[1].astype(_U),
                                    c1[2].astype(_U), c1[3].astype(_U))
                h1 = h1 % _U(ROWS * DIM)
                sl = pl.ds(j * LANES, LANES)
                h0r_v[c, p, sl] = (h0 >> _U(1)).astype(jnp.int32)
                h0c_v[c, p, sl] = ((h0 & _U(1)) << _U(6)).astype(jnp.int32)
                h1r_v[c, p, sl] = (h1 >> _U(7)).astype(jnp.int32)
                h1c_v[c, p, sl] = (h1 & _U(VDIM - 1)).astype(jnp.int32)
            return _

        lax.fori_loop(jnp.int32(0), jnp.int32(PIECE // LANES), body,
                      jnp.int32(0))

    for p in range(N_PIECE):
        hash_piece(p)

    sem_r = [sem_r0, sem_r1]
    sem_o = [sem_o0, sem_o1]

    def fire(p):
        buf = p % 2
        cps = []
        for c in range(N_CHUNKS):
            cps.append(pltpu.async_copy(view_hbm.at[h0r_v.at[c, p]],
                                        rows_v.at[buf, c], sem_r[buf]))
            cps.append(pltpu.async_copy(view_hbm.at[h1r_v.at[c, p]],
                                        wrows_v.at[buf, c], sem_r[buf]))
        return cps

    cps = fire(0)
    out_cps = [None, None]
    lane = lax.iota(jnp.int32, LANES)
    for p in range(N_PIECE):
        buf = p % 2
        for cp in cps:
            cp.wait()
        if p + 1 < N_PIECE:
            cps = fire(p + 1)
        if out_cps[buf] is not None:
            out_cps[buf].wait()
            out_cps[buf] = None

        bufv = jnp.full((LANES,), buf, jnp.int32)
        for c in range(N_CHUNKS):
            cv = jnp.full((LANES,), c, jnp.int32)
            for g in range(PIECE // LANES):
                cols = h1c_v[c, p, pl.ds(g * LANES, LANES)]
                wv = plsc.load_gather(
                    wrows_v, [bufv, cv, lane + jnp.int32(g * LANES), cols])
                w_v[c, pl.ds(p * PIECE + g * LANES, LANES)] = wv * SCALE

        pv = jnp.full((LANES,), p, jnp.int32)

        def body(b, _):
            bidx = jnp.full((LANES,), p * PIECE + b, jnp.int32)
            bv_ = jnp.full((LANES,), b, jnp.int32)
            cvs = [jnp.full((LANES,), c, jnp.int32) for c in range(N_CHUNKS)]
            wv = [plsc.load_gather(w_v, [cvs[c], bidx])
                  for c in range(N_CHUNKS)]
            cb = [plsc.load_gather(h0c_v, [cvs[c], pv, bv_]) + lane
                  for c in range(N_CHUNKS)]
            for d in range(DIM // LANES):
                sl = pl.ds(d * LANES, LANES)
                acc = None
                for c in range(N_CHUNKS):
                    rv = plsc.load_gather(
                        rows_v,
                        [bufv, cvs[c], bv_, cb[c] + jnp.int32(d * LANES)])
                    acc = wv[c] * rv if acc is None else acc + wv[c] * rv
                out_v[buf, b, sl] = acc
            return _

        lax.fori_loop(jnp.int32(0), jnp.int32(PIECE), body, jnp.int32(0))
        out_cps[buf] = pltpu.async_copy(
            out_v.at[buf], out_hbm.at[pl.ds(base + p * PIECE, PIECE)],
            sem_o[buf])

    for cp in out_cps:
        if cp is not None:
            cp.wait()


@jax.jit
def _wh_embed(x_i32, view, params):
    mesh = plsc.VectorSubcoreMesh(core_axis_name="c", subcore_axis_name="s")
    f = pl.kernel(
        _sc_body,
        out_type=jax.ShapeDtypeStruct((BATCH, DIM), jnp.float32),
        mesh=mesh,
        scratch_types=[
            pltpu.VMEM((B_PER_W,), jnp.int32),
            pltpu.VMEM((2, N_CHUNKS, 4, LANES), jnp.int32),
            pltpu.VMEM((N_CHUNKS, N_PIECE, PIECE), jnp.int32),
            pltpu.VMEM((N_CHUNKS, N_PIECE, PIECE), jnp.int32),
            pltpu.VMEM((N_CHUNKS, N_PIECE, PIECE), jnp.int32),
            pltpu.VMEM((N_CHUNKS, N_PIECE, PIECE), jnp.int32),
            pltpu.VMEM((N_CHUNKS, B_PER_W), jnp.float32),
            pltpu.VMEM((2, N_CHUNKS, PIECE, VDIM), jnp.float32),
            pltpu.VMEM((2, N_CHUNKS, PIECE, VDIM), jnp.float32),
            pltpu.VMEM((2, PIECE, DIM), jnp.float32),
            pltpu.SemaphoreType.DMA,
            pltpu.SemaphoreType.DMA,
            pltpu.SemaphoreType.DMA,
            pltpu.SemaphoreType.DMA,
        ],
        compiler_params=pltpu.CompilerParams(needs_layout_passes=False),
    )
    return f(x_i32, view, params)


def kernel(x, table, a0, b0, a1, b1):
    x_i32 = x.astype(jnp.int32)
    p = []
    for a, b in ((a0, b0), (a1, b1)):
        a = a.astype(jnp.int64)
        p.append(jnp.stack([a >> 22, (a >> 11) & 2047, a & 2047,
                            b.astype(jnp.int64)], axis=-1))
    params = jnp.stack(p).astype(jnp.int32)
    params = jnp.broadcast_to(params[..., None],
                              (2, N_CHUNKS, 4, LANES))
    with jax.enable_x64(False):
        view = _format(jnp.swapaxes(table, 0, 1))
        return _wh_embed(x_i32, view, params)


# speedup vs baseline: 2.4201x; 1.8095x over previous
import jax
import jax.numpy as jnp
from jax import lax
from jax.experimental import pallas as pl
from jax.experimental.pallas import tpu as pltpu
from jax.experimental.pallas import tpu_sc as plsc

ROWS = 1000000
DIM = 64
N_CHUNKS = 4
BATCH = 16384
PRIME = (1 << 31) - 1

NC = 2
NS = 16
LANES = 16
NW = NC * NS
B_PER_W = BATCH // NW
PIECE = 64
N_PIECE = B_PER_W // PIECE
SCALE = (N_CHUNKS * DIM) ** 0.5 / N_CHUNKS

_U = jnp.uint32


def _modp(r):
    return jnp.minimum(r, r - _U(PRIME))


def _shift_modp(v, k):
    lo = (v << _U(k)) & _U(PRIME)
    hi = v >> _U(31 - k)
    return _modp(lo + hi)


def _poly_hash_vec(xv, a2, a1, a0, bv):
    t2 = _shift_modp(_modp(a2 * xv), 22)
    t1 = _shift_modp(_modp(a1 * xv), 11)
    t0 = _modp(a0 * xv)
    s = _modp(t2 + t1)
    s = _modp(s + t0)
    return _modp(s + bv)


def _sc_body(x_hbm, table_hbm, params_hbm, out_hbm,
             x_v, params_v, h0_v, h1r_v, h1c_v, w_v, rows_v, wrows_v,
             out_v, sem_r0, sem_r1, sem_o0, sem_o1):
    wid = lax.axis_index("s") * NC + lax.axis_index("c")
    base = wid * B_PER_W

    pltpu.sync_copy(x_hbm.at[pl.ds(base, B_PER_W)], x_v)
    pltpu.sync_copy(params_hbm, params_v)

    coef = [[[params_v[f, c, j] for j in range(4)]
             for c in range(N_CHUNKS)] for f in range(2)]

    def hash_piece(p):
        def body(j, _):
            xv = x_v[pl.ds(p * PIECE + j * LANES, LANES)].astype(_U)
            for c in range(N_CHUNKS):
                c0 = coef[0][c]
                h0 = _poly_hash_vec(xv, c0[0].astype(_U), c0[1].astype(_U),
                                    c0[2].astype(_U), c0[3].astype(_U))
                h0 = h0 % _U(ROWS)
                c1 = coef[1][c]
                h1 = _poly_hash_vec(xv, c1[0].astype(_U), c1[1].astype(_U),
                                    c1[2].astype(_U), c1[3].astype(_U))
                h1 = h1 % _U(ROWS * DIM)
                sl = pl.ds(j * LANES, LANES)
                h0_v[c, p, sl] = h0.astype(jnp.int32)
                h1r_v[c, p, sl] = (h1 >> _U(6)).astype(jnp.int32)
                h1c_v[c, p, sl] = (h1 & _U(DIM - 1)).astype(jnp.int32)
            return _

        lax.fori_loop(jnp.int32(0), jnp.int32(PIECE // LANES), body,
                      jnp.int32(0))

    for p in range(N_PIECE):
        hash_piece(p)

    sem_r = [sem_r0, sem_r1]
    sem_o = [sem_o0, sem_o1]

    def fire(p):
        buf = p % 2
        cps = []
        for c in range(N_CHUNKS):
            cps.append(pltpu.async_copy(table_hbm.at[h0_v.at[c, p]],
                                        rows_v.at[buf, c], sem_r[buf]))
            cps.append(pltpu.async_copy(table_hbm.at[h1r_v.at[c, p]],
                                        wrows_v.at[buf, c], sem_r[buf]))
        return cps

    cps = fire(0)
    out_cps = [None, None]
    lane = lax.iota(jnp.int32, LANES)
    for p in range(N_PIECE):
        buf = p % 2
        for cp in cps:
            cp.wait()
        if p + 1 < N_PIECE:
            cps = fire(p + 1)
        if out_cps[buf] is not None:
            out_cps[buf].wait()
            out_cps[buf] = None

        bufv = jnp.full((LANES,), buf, jnp.int32)
        for c in range(N_CHUNKS):
            cv = jnp.full((LANES,), c, jnp.int32)
            for g in range(PIECE // LANES):
                cols = h1c_v[c, p, pl.ds(g * LANES, LANES)]
                wv = plsc.load_gather(
                    wrows_v, [bufv, cv, lane + jnp.int32(g * LANES), cols])
                w_v[c, pl.ds(p * PIECE + g * LANES, LANES)] = wv * SCALE

        def body(b, _):
            bidx = jnp.full((LANES,), p * PIECE + b, jnp.int32)
            wv = [plsc.load_gather(
                      w_v, [jnp.full((LANES,), c, jnp.int32), bidx])
                  for c in range(N_CHUNKS)]
            for d in range(DIM // LANES):
                sl = pl.ds(d * LANES, LANES)
                acc = wv[0] * rows_v[buf, 0, b, sl]
                for c in range(1, N_CHUNKS):
                    acc = acc + wv[c] * rows_v[buf, c, b, sl]
                out_v[buf, b, sl] = acc
            return _

        lax.fori_loop(jnp.int32(0), jnp.int32(PIECE), body, jnp.int32(0))
        out_cps[buf] = pltpu.async_copy(
            out_v.at[buf], out_hbm.at[pl.ds(base + p * PIECE, PIECE)],
            sem_o[buf])

    for cp in out_cps:
        if cp is not None:
            cp.wait()


@jax.jit
def _wh_embed(x_i32, table, params):
    mesh = plsc.VectorSubcoreMesh(core_axis_name="c", subcore_axis_name="s")
    f = pl.kernel(
        _sc_body,
        out_type=jax.ShapeDtypeStruct((BATCH, DIM), jnp.float32),
        mesh=mesh,
        scratch_types=[
            pltpu.VMEM((B_PER_W,), jnp.int32),
            pltpu.VMEM((2, N_CHUNKS, 4, LANES), jnp.int32),
            pltpu.VMEM((N_CHUNKS, N_PIECE, PIECE), jnp.int32),
            pltpu.VMEM((N_CHUNKS, N_PIECE, PIECE), jnp.int32),
            pltpu.VMEM((N_CHUNKS, N_PIECE, PIECE), jnp.int32),
            pltpu.VMEM((N_CHUNKS, B_PER_W), jnp.float32),
            pltpu.VMEM((2, N_CHUNKS, PIECE, DIM), jnp.float32),
            pltpu.VMEM((2, N_CHUNKS, PIECE, DIM), jnp.float32),
            pltpu.VMEM((2, PIECE, DIM), jnp.float32),
            pltpu.SemaphoreType.DMA,
            pltpu.SemaphoreType.DMA,
            pltpu.SemaphoreType.DMA,
            pltpu.SemaphoreType.DMA,
        ],
        compiler_params=pltpu.CompilerParams(needs_layout_passes=False,
                                             use_tc_tiling_on_sc=False),
    )
    return f(x_i32, table, params)


def kernel(x, table, a0, b0, a1, b1):
    x_i32 = x.astype(jnp.int32)
    p = []
    for a, b in ((a0, b0), (a1, b1)):
        a = a.astype(jnp.int64)
        p.append(jnp.stack([a >> 22, (a >> 11) & 2047, a & 2047,
                            b.astype(jnp.int64)], axis=-1))
    params = jnp.stack(p).astype(jnp.int32)
    params = jnp.broadcast_to(params[..., None],
                              (2, N_CHUNKS, 4, LANES))
    with jax.enable_x64(False):
        return _wh_embed(x_i32, table, params)
